# baseline (speedup 1.0000x reference)
import functools

import jax
import jax.numpy as jnp
from jax import lax
from jax.experimental import pallas as pl
from jax.experimental.pallas import tpu as pltpu
from jax.experimental.pallas import tpu_sc as plsc

B, N, S, K = 8, 8192, 512, 24
NC, NS = 2, 16
NW = NC * NS
TW = 16
ROWS = B * S
GTOT = K * ROWS + ROWS
CHUNK = GTOT // NW


def _fps_body(x_ref, y_ref, z_ref, cid_ref, cx_ref, cy_ref, cz_ref, dist_ref):
    col = lax.broadcasted_iota(jnp.int32, (B, N), 1)
    col_s = lax.broadcasted_iota(jnp.int32, (B, S), 1)
    x = x_ref[...]
    y = y_ref[...]
    z = z_ref[...]
    dist_ref[...] = jnp.full((B, N), 1e10, jnp.float32)
    cid_ref[...] = jnp.zeros((B, S), jnp.int32)
    cx_ref[...] = jnp.zeros((B, S), jnp.float32)
    cy_ref[...] = jnp.zeros((B, S), jnp.float32)
    cz_ref[...] = jnp.zeros((B, S), jnp.float32)

    def body(i, f):
        msk = col == f
        cx = jnp.sum(jnp.where(msk, x, 0.0), axis=1, keepdims=True)
        cy = jnp.sum(jnp.where(msk, y, 0.0), axis=1, keepdims=True)
        cz = jnp.sum(jnp.where(msk, z, 0.0), axis=1, keepdims=True)
        sel = col_s == i
        cid_ref[...] = jnp.where(sel, jnp.broadcast_to(f, (B, S)), cid_ref[...])
        cx_ref[...] = jnp.where(sel, jnp.broadcast_to(cx, (B, S)), cx_ref[...])
        cy_ref[...] = jnp.where(sel, jnp.broadcast_to(cy, (B, S)), cy_ref[...])
        cz_ref[...] = jnp.where(sel, jnp.broadcast_to(cz, (B, S)), cz_ref[...])
        dx = x - cx
        dy = y - cy
        dz = z - cz
        dd = (dx * dx + dy * dy) + dz * dz
        dist = jnp.minimum(dist_ref[...], dd)
        dist_ref[...] = dist
        m = jnp.max(dist, axis=1, keepdims=True)
        f_new = jnp.min(jnp.where(dist == m, col, N), axis=1, keepdims=True)
        return f_new

    lax.fori_loop(0, S, body, jnp.zeros((B, 1), jnp.int32))


def _run_fps(x, y, z):
    return pl.pallas_call(
        _fps_body,
        out_shape=(
            jax.ShapeDtypeStruct((B, S), jnp.int32),
            jax.ShapeDtypeStruct((B, S), jnp.float32),
            jax.ShapeDtypeStruct((B, S), jnp.float32),
            jax.ShapeDtypeStruct((B, S), jnp.float32),
        ),
        scratch_shapes=[pltpu.VMEM((B, N), jnp.float32)],
    )(x, y, z)


TK_ROWS = 64


NLC = N // 128
LC = 5
INF = float("inf")
BIGI = 2 * N


def _topk_body(x_ref, y_ref, z_ref, cx_ref, cy_ref, cz_ref, out_ref,
               dsc_ref):
    b = pl.program_id(0)
    lane8 = lax.broadcasted_iota(jnp.int32, (TK_ROWS, 128), 1)
    bsel = lane8 == b

    def pick(ref):
        return jnp.sum(jnp.where(bsel, ref[...], 0.0), axis=1, keepdims=True)

    cxs = pick(cx_ref)
    cys = pick(cy_ref)
    czs = pick(cz_ref)
    lane = lax.broadcasted_iota(jnp.int32, (8, 128), 1)
    col_k = lax.broadcasted_iota(jnp.int32, (8, K), 1)
    overflow = jnp.int32(0)

    for rg in range(TK_ROWS // 8):
        cxg = cxs[rg * 8:(rg + 1) * 8]
        cyg = cys[rg * 8:(rg + 1) * 8]
        czg = czs[rg * 8:(rg + 1) * 8]

        def stream(j, lad):
            xj = x_ref[0, pl.ds(j, 1), :]
            yj = y_ref[0, pl.ds(j, 1), :]
            zj = z_ref[0, pl.ds(j, 1), :]
            dx = cxg - xj
            dy = cyg - yj
            dz = czg - zj
            v = (dx * dx + dy * dy) + dz * dz
            iv = lane + j * 128
            new = []
            for l in range(LC):
                V, I = lad[2 * l], lad[2 * l + 1]
                take = v < V
                new.append(jnp.where(take, v, V))
                new.append(jnp.where(take, iv, I))
                v, iv = jnp.where(take, V, v), jnp.where(take, I, iv)
            return tuple(new)

        init = []
        for l in range(LC):
            init.append(jnp.full((8, 128), INF, jnp.float32))
            init.append(jnp.full((8, 128), BIGI, jnp.int32))
        lad = lax.fori_loop(0, NLC, stream, tuple(init))

        V = [lad[2 * l] for l in range(LC)]
        I = [lad[2 * l + 1] for l in range(LC)]
        pops = jnp.zeros((8, 128), jnp.int32)
        acc = jnp.zeros((8, K), jnp.int32)
        for k in range(K):
            m = jnp.min(V[0], axis=1, keepdims=True)
            cand = V[0] == m
            idx = jnp.min(jnp.where(cand, I[0], BIGI), axis=1, keepdims=True)
            hit = cand & (I[0] == idx)
            acc = jnp.where(col_k == k, jnp.broadcast_to(idx, (8, K)), acc)
            for l in range(LC - 1):
                V[l] = jnp.where(hit, V[l + 1], V[l])
                I[l] = jnp.where(hit, I[l + 1], I[l])
            V[LC - 1] = jnp.where(hit, INF, V[LC - 1])
            I[LC - 1] = jnp.where(hit, BIGI, I[LC - 1])
            pops = pops + hit.astype(jnp.int32)
        out_ref[0, rg * 8:(rg + 1) * 8, :] = acc
        overflow = jnp.maximum(overflow, jnp.max(pops))

    @pl.when(overflow >= LC)
    def _recover():
        col3 = (lax.broadcasted_iota(jnp.int32, (TK_ROWS, NLC, 128), 1) * 128
                + lax.broadcasted_iota(jnp.int32, (TK_ROWS, NLC, 128), 2))

        def fill(j, _):
            dx = cxs - x_ref[0, pl.ds(j, 1), :]
            dy = cys - y_ref[0, pl.ds(j, 1), :]
            dz = czs - z_ref[0, pl.ds(j, 1), :]
            dsc_ref[:, pl.ds(j, 1), :] = (
                (dx * dx + dy * dy) + dz * dz)[:, None, :]
            return 0

        lax.fori_loop(0, NLC, fill, 0)
        d = dsc_ref[...]
        col_kf = lax.broadcasted_iota(jnp.int32, (TK_ROWS, K), 1)
        accf = jnp.zeros((TK_ROWS, K), jnp.int32)
        for k in range(K):
            m2 = jnp.min(jnp.min(d, axis=2), axis=1)[:, None, None]
            idx = jnp.min(jnp.min(jnp.where(d == m2, col3, BIGI), axis=2),
                          axis=1)[:, None]
            accf = jnp.where(col_kf == k,
                             jnp.broadcast_to(idx, (TK_ROWS, K)), accf)
            d = jnp.where(col3 == idx[:, :, None], INF, d)
        out_ref[0] = accf


def _run_topk(x, y, z, cxt, cyt, czt):
    grid = (B, S // TK_ROWS)
    pspec = pl.BlockSpec((1, NLC, 128), lambda b, j: (b, 0, 0))
    cspec = pl.BlockSpec((TK_ROWS, 128), lambda b, j: (j, 0))
    return pl.pallas_call(
        _topk_body,
        grid=grid,
        in_specs=[pspec, pspec, pspec, cspec, cspec, cspec],
        out_specs=pl.BlockSpec((1, TK_ROWS, K), lambda b, j: (b, j, 0)),
        out_shape=jax.ShapeDtypeStruct((B, S, K), jnp.int32),
        scratch_shapes=[pltpu.VMEM((TK_ROWS, NLC, 128), jnp.float32)],
    )(x, y, z, cxt, cyt, czt)


NB_CHUNK = K * ROWS // NW
CE_CHUNK = ROWS // NW


def _gather_rows(table, gnb, gce):
    mesh = plsc.VectorSubcoreMesh(core_axis_name="c", subcore_axis_name="s")

    @functools.partial(
        pl.kernel,
        mesh=mesh,
        compiler_params=pltpu.CompilerParams(use_tc_tiling_on_sc=False),
        out_type=(
            jax.ShapeDtypeStruct((K * ROWS, TW), jnp.float32),
            jax.ShapeDtypeStruct((ROWS, TW), jnp.float32),
        ),
        scratch_types=[
            pltpu.VMEM((NB_CHUNK,), jnp.int32),
            pltpu.VMEM((NB_CHUNK, TW), jnp.float32),
            pltpu.VMEM((CE_CHUNK,), jnp.int32),
            pltpu.VMEM((CE_CHUNK, TW), jnp.float32),
            pltpu.SemaphoreType.DMA,
        ],
    )
    def gk(table_hbm, gnb_hbm, gce_hbm, nb_hbm, ce_hbm,
           nbi_v, nbr_v, cei_v, cer_v, sem):
        wid = lax.axis_index("s") * NC + lax.axis_index("c")
        nbase = wid * NB_CHUNK
        cbase = wid * CE_CHUNK
        pltpu.sync_copy(gnb_hbm.at[pl.ds(nbase, NB_CHUNK)], nbi_v)
        pltpu.sync_copy(gce_hbm.at[pl.ds(cbase, CE_CHUNK)], cei_v)
        pltpu.async_copy(table_hbm.at[nbi_v], nbr_v, sem).wait()
        pltpu.async_copy(table_hbm.at[cei_v], cer_v, sem).wait()
        pltpu.sync_copy(nbr_v, nb_hbm.at[pl.ds(nbase, NB_CHUNK)])
        pltpu.sync_copy(cer_v, ce_hbm.at[pl.ds(cbase, CE_CHUNK)])

    return gk(table, gnb, gce)


MR = 128


def _mlp_body(nb_ref, ce_ref, w1_ref, c1_ref, b1_ref, w2_ref,
              wa_ref, ba_ref, wb_ref, wc_ref, bc_ref, wd_ref,
              we_ref, be_ref, wf_ref, b2_ref, out_ref):
    nb = nb_ref[...]
    ce = ce_ref[...]
    nbf = nb.reshape(K * MR, TW)

    def branch(w1, corrw, b1, w2, kpref):
        h = jnp.dot(nbf, w1, preferred_element_type=jnp.float32)
        corr = jnp.dot(ce, corrw, preferred_element_type=jnp.float32)
        bias = b1 - corr
        h3 = h.reshape(K, MR, 128) + bias[None, :, :]
        h3 = jnp.maximum(h3, 0.0)
        g = jnp.dot(h3.reshape(K * MR, 128), w2,
                    preferred_element_type=jnp.float32)
        g3 = g.reshape(K, MR, 128)
        niota = lax.broadcasted_iota(jnp.int32, (K, MR, 128), 0)
        return jnp.max(jnp.where(niota < kpref, g3, -jnp.inf), axis=0)

    outs = (
        branch(w1_ref[...], c1_ref[...], b1_ref[...], w2_ref[...], 16),
        branch(wa_ref[...], wa_ref[...], ba_ref[...], wb_ref[...], 8),
        branch(wc_ref[...], wc_ref[...], bc_ref[...], wd_ref[...], 16),
        branch(we_ref[...], we_ref[...], be_ref[...], wf_ref[...], 24),
    )
    for i, o in enumerate(outs):
        out_ref[0, i * 128:(i + 1) * 128, :] = jnp.transpose(
            o + b2_ref[i:i + 1, :])


def _run_mlp(nb, ce, weights, b2all):
    grid = (ROWS // MR,)
    wspec = pl.BlockSpec((TW, 128), lambda t: (0, 0))
    w2spec = pl.BlockSpec((128, 128), lambda t: (0, 0))
    bspec = pl.BlockSpec((1, 128), lambda t: (0, 0))
    spb = S // MR
    return pl.pallas_call(
        _mlp_body,
        grid=grid,
        in_specs=[
            pl.BlockSpec((K, MR, TW), lambda t: (0, t, 0)),
            pl.BlockSpec((MR, TW), lambda t: (t, 0)),
            wspec, wspec, bspec, w2spec,
            wspec, bspec, w2spec,
            wspec, bspec, w2spec,
            wspec, bspec, w2spec,
            pl.BlockSpec((4, 128), lambda t: (0, 0)),
        ],
        out_specs=pl.BlockSpec((1, 512, MR),
                               lambda t: (t // spb, 0, t % spb)),
        out_shape=jax.ShapeDtypeStruct((B, 512, S), jnp.float32),
    )(nb, ce, *weights, b2all)


def kernel(l0_xyz, l0_points, sa_W1, sa_b1, sa_W2, sa_b2,
           ms_W1_0, ms_b1_0, ms_W2_0, ms_b2_0,
           ms_W1_1, ms_b1_1, ms_W2_1, ms_b2_1,
           ms_W1_2, ms_b1_2, ms_W2_2, ms_b2_2):
    x = l0_xyz[:, 0, :]
    y = l0_xyz[:, 1, :]
    z = l0_xyz[:, 2, :]

    cid, cx, cy, cz = _run_fps(x, y, z)

    def padT(a):
        return jnp.pad(a.T, ((0, 0), (0, 128 - B)))

    x3 = x.reshape(B, NLC, 128)
    y3 = y.reshape(B, NLC, 128)
    z3 = z.reshape(B, NLC, 128)
    idx24 = _run_topk(x3, y3, z3, padT(cx), padT(cy), padT(cz))

    pts = jnp.transpose(l0_points, (0, 2, 1)).reshape(B * N, 3)
    xyzt = jnp.transpose(l0_xyz, (0, 2, 1)).reshape(B * N, 3)
    table = jnp.concatenate(
        [xyzt, pts, jnp.zeros((B * N, TW - 6), jnp.float32)], axis=1)

    boff = (jnp.arange(B, dtype=jnp.int32) * N)[:, None, None]
    gnb = jnp.transpose(idx24 + boff, (2, 0, 1)).reshape(K * ROWS)
    gce = (cid + boff[:, :, 0]).reshape(ROWS)

    nb_flat, ce = _gather_rows(table, gnb, gce)
    nb = nb_flat.reshape(K, ROWS, TW)

    def padw1(w, r0):
        out = jnp.zeros((TW, 128), jnp.float32)
        return lax.dynamic_update_slice(out, w, (r0, 0))

    w1_base = padw1(sa_W1, 0)
    c1_base = padw1(sa_W1[:3], 0)
    w2_base = jnp.zeros((128, 128), jnp.float32).at[:64].set(sa_W2)
    b1_base = jnp.zeros((1, 128), jnp.float32).at[0, :64].set(sa_b1)

    def branch_w(W1, b1, W2):
        return (padw1(W1, 0),
                jnp.zeros((1, 128), jnp.float32).at[0, :64].set(b1),
                jnp.zeros((128, 128), jnp.float32).at[:64].set(W2))

    wa, ba, wb = branch_w(ms_W1_0, ms_b1_0, ms_W2_0)
    wc, bc, wd = branch_w(ms_W1_1, ms_b1_1, ms_W2_1)
    we, be, wf = branch_w(ms_W1_2, ms_b1_2, ms_W2_2)

    b2all = jnp.stack([sa_b2, ms_b2_0, ms_b2_1, ms_b2_2], axis=0)
    feats = _run_mlp(
        nb, ce,
        (w1_base, c1_base, b1_base, w2_base,
         wa, ba, wb, wc, bc, wd, we, be, wf),
        b2all)

    keypoints = jnp.stack([cx, cy, cz], axis=1)
    return (keypoints, feats)


# speedup vs baseline: 3.2584x; 3.2584x over previous
import functools

import jax
import jax.numpy as jnp
from jax import lax
from jax.experimental import pallas as pl
from jax.experimental.pallas import tpu as pltpu
from jax.experimental.pallas import tpu_sc as plsc

B, N, S, K = 8, 8192, 512, 24
NC, NS = 2, 16
NW = NC * NS
TW = 16
ROWS = B * S
GTOT = K * ROWS + ROWS
CHUNK = GTOT // NW


def _fps_body(x_ref, y_ref, z_ref, cid_ref, cx_ref, cy_ref, cz_ref, dist_ref):
    col = lax.broadcasted_iota(jnp.int32, (B, N), 1)
    col_s = lax.broadcasted_iota(jnp.int32, (B, S), 1)
    x = x_ref[...]
    y = y_ref[...]
    z = z_ref[...]
    dist_ref[...] = jnp.full((B, N), 1e10, jnp.float32)
    cid_ref[...] = jnp.zeros((B, S), jnp.int32)
    cx_ref[...] = jnp.zeros((B, S), jnp.float32)
    cy_ref[...] = jnp.zeros((B, S), jnp.float32)
    cz_ref[...] = jnp.zeros((B, S), jnp.float32)

    def body(i, f):
        msk = col == f
        cx = jnp.sum(jnp.where(msk, x, 0.0), axis=1, keepdims=True)
        cy = jnp.sum(jnp.where(msk, y, 0.0), axis=1, keepdims=True)
        cz = jnp.sum(jnp.where(msk, z, 0.0), axis=1, keepdims=True)
        sel = col_s == i
        cid_ref[...] = jnp.where(sel, jnp.broadcast_to(f, (B, S)), cid_ref[...])
        cx_ref[...] = jnp.where(sel, jnp.broadcast_to(cx, (B, S)), cx_ref[...])
        cy_ref[...] = jnp.where(sel, jnp.broadcast_to(cy, (B, S)), cy_ref[...])
        cz_ref[...] = jnp.where(sel, jnp.broadcast_to(cz, (B, S)), cz_ref[...])
        dx = x - cx
        dy = y - cy
        dz = z - cz
        dd = (dx * dx + dy * dy) + dz * dz
        dist = jnp.minimum(dist_ref[...], dd)
        dist_ref[...] = dist
        m = jnp.max(dist, axis=1, keepdims=True)
        f_new = jnp.min(jnp.where(dist == m, col, N), axis=1, keepdims=True)
        return f_new

    lax.fori_loop(0, S, body, jnp.zeros((B, 1), jnp.int32))


def _run_fps(x, y, z):
    return pl.pallas_call(
        _fps_body,
        out_shape=(
            jax.ShapeDtypeStruct((B, S), jnp.int32),
            jax.ShapeDtypeStruct((B, S), jnp.float32),
            jax.ShapeDtypeStruct((B, S), jnp.float32),
            jax.ShapeDtypeStruct((B, S), jnp.float32),
        ),
        scratch_shapes=[pltpu.VMEM((B, N), jnp.float32)],
    )(x, y, z)


TK_ROWS = 64


NLC = N // 128
LC = 5
INF = float("inf")
BIGI = 2 * N


def _topk_body(x_ref, y_ref, z_ref, cx_ref, cy_ref, cz_ref, out_ref,
               dsc_ref):
    b = pl.program_id(0)
    lane8 = lax.broadcasted_iota(jnp.int32, (TK_ROWS, 128), 1)
    bsel = lane8 == b

    def pick(ref):
        return jnp.sum(jnp.where(bsel, ref[...], 0.0), axis=1, keepdims=True)

    cxs = pick(cx_ref)
    cys = pick(cy_ref)
    czs = pick(cz_ref)
    lane = lax.broadcasted_iota(jnp.int32, (8, 128), 1)
    col_k = lax.broadcasted_iota(jnp.int32, (8, K), 1)
    overflow = jnp.int32(0)

    for rg in range(TK_ROWS // 8):
        cxg = cxs[rg * 8:(rg + 1) * 8]
        cyg = cys[rg * 8:(rg + 1) * 8]
        czg = czs[rg * 8:(rg + 1) * 8]

        V = [jnp.full((8, 128), INF, jnp.float32) for _ in range(LC)]
        I = [jnp.full((8, 128), BIGI, jnp.int32) for _ in range(LC)]
        for j in range(NLC):
            dx = cxg - x_ref[0, j, :][None, :]
            dy = cyg - y_ref[0, j, :][None, :]
            dz = czg - z_ref[0, j, :][None, :]
            v = (dx * dx + dy * dy) + dz * dz
            iv = lane + j * 128
            for l in range(LC):
                take = v < V[l]
                nv = jnp.where(take, v, V[l])
                ni = jnp.where(take, iv, I[l])
                v, iv = jnp.where(take, V[l], v), jnp.where(take, I[l], iv)
                V[l], I[l] = nv, ni
        pops = jnp.zeros((8, 128), jnp.int32)
        acc = jnp.zeros((8, K), jnp.int32)
        for k in range(K):
            m = jnp.min(V[0], axis=1, keepdims=True)
            cand = V[0] == m
            idx = jnp.min(jnp.where(cand, I[0], BIGI), axis=1, keepdims=True)
            hit = cand & (I[0] == idx)
            acc = jnp.where(col_k == k, jnp.broadcast_to(idx, (8, K)), acc)
            for l in range(LC - 1):
                V[l] = jnp.where(hit, V[l + 1], V[l])
                I[l] = jnp.where(hit, I[l + 1], I[l])
            V[LC - 1] = jnp.where(hit, INF, V[LC - 1])
            I[LC - 1] = jnp.where(hit, BIGI, I[LC - 1])
            pops = pops + hit.astype(jnp.int32)
        out_ref[0, rg * 8:(rg + 1) * 8, :] = acc
        overflow = jnp.maximum(overflow, jnp.max(pops))

    @pl.when(overflow >= LC)
    def _recover():
        col3 = (lax.broadcasted_iota(jnp.int32, (TK_ROWS, NLC, 128), 1) * 128
                + lax.broadcasted_iota(jnp.int32, (TK_ROWS, NLC, 128), 2))

        def fill(j, _):
            dx = cxs - x_ref[0, pl.ds(j, 1), :]
            dy = cys - y_ref[0, pl.ds(j, 1), :]
            dz = czs - z_ref[0, pl.ds(j, 1), :]
            dsc_ref[:, pl.ds(j, 1), :] = (
                (dx * dx + dy * dy) + dz * dz)[:, None, :]
            return 0

        lax.fori_loop(0, NLC, fill, 0)
        d = dsc_ref[...]
        col_kf = lax.broadcasted_iota(jnp.int32, (TK_ROWS, K), 1)
        accf = jnp.zeros((TK_ROWS, K), jnp.int32)
        for k in range(K):
            m2 = jnp.min(jnp.min(d, axis=2), axis=1)[:, None, None]
            idx = jnp.min(jnp.min(jnp.where(d == m2, col3, BIGI), axis=2),
                          axis=1)[:, None]
            accf = jnp.where(col_kf == k,
                             jnp.broadcast_to(idx, (TK_ROWS, K)), accf)
            d = jnp.where(col3 == idx[:, :, None], INF, d)
        out_ref[0] = accf


def _run_topk(x, y, z, cxt, cyt, czt):
    grid = (B, S // TK_ROWS)
    pspec = pl.BlockSpec((1, NLC, 128), lambda b, j: (b, 0, 0))
    cspec = pl.BlockSpec((TK_ROWS, 128), lambda b, j: (j, 0))
    return pl.pallas_call(
        _topk_body,
        grid=grid,
        in_specs=[pspec, pspec, pspec, cspec, cspec, cspec],
        out_specs=pl.BlockSpec((1, TK_ROWS, K), lambda b, j: (b, j, 0)),
        out_shape=jax.ShapeDtypeStruct((B, S, K), jnp.int32),
        scratch_shapes=[pltpu.VMEM((TK_ROWS, NLC, 128), jnp.float32)],
    )(x, y, z, cxt, cyt, czt)


NB_CHUNK = K * ROWS // NW
CE_CHUNK = ROWS // NW


def _gather_rows(table, gnb, gce):
    mesh = plsc.VectorSubcoreMesh(core_axis_name="c", subcore_axis_name="s")

    @functools.partial(
        pl.kernel,
        mesh=mesh,
        compiler_params=pltpu.CompilerParams(use_tc_tiling_on_sc=False),
        out_type=(
            jax.ShapeDtypeStruct((K * ROWS, TW), jnp.float32),
            jax.ShapeDtypeStruct((ROWS, TW), jnp.float32),
        ),
        scratch_types=[
            pltpu.VMEM((NB_CHUNK,), jnp.int32),
            pltpu.VMEM((NB_CHUNK, TW), jnp.float32),
            pltpu.VMEM((CE_CHUNK,), jnp.int32),
            pltpu.VMEM((CE_CHUNK, TW), jnp.float32),
            pltpu.SemaphoreType.DMA,
        ],
    )
    def gk(table_hbm, gnb_hbm, gce_hbm, nb_hbm, ce_hbm,
           nbi_v, nbr_v, cei_v, cer_v, sem):
        wid = lax.axis_index("s") * NC + lax.axis_index("c")
        nbase = wid * NB_CHUNK
        cbase = wid * CE_CHUNK
        pltpu.sync_copy(gnb_hbm.at[pl.ds(nbase, NB_CHUNK)], nbi_v)
        pltpu.sync_copy(gce_hbm.at[pl.ds(cbase, CE_CHUNK)], cei_v)
        pltpu.async_copy(table_hbm.at[nbi_v], nbr_v, sem).wait()
        pltpu.async_copy(table_hbm.at[cei_v], cer_v, sem).wait()
        pltpu.sync_copy(nbr_v, nb_hbm.at[pl.ds(nbase, NB_CHUNK)])
        pltpu.sync_copy(cer_v, ce_hbm.at[pl.ds(cbase, CE_CHUNK)])

    return gk(table, gnb, gce)


MR = 128


def _mlp_body(nb_ref, ce_ref, w1_ref, c1_ref, b1_ref, w2_ref,
              wa_ref, ba_ref, wb_ref, wc_ref, bc_ref, wd_ref,
              we_ref, be_ref, wf_ref, b2_ref, out_ref):
    nb = nb_ref[...]
    ce = ce_ref[...]
    nbf = nb.reshape(K * MR, TW)

    def branch(w1, corrw, b1, w2, kpref):
        h = jnp.dot(nbf, w1, preferred_element_type=jnp.float32)
        corr = jnp.dot(ce, corrw, preferred_element_type=jnp.float32)
        bias = b1 - corr
        h3 = h.reshape(K, MR, 128) + bias[None, :, :]
        h3 = jnp.maximum(h3, 0.0)
        g = jnp.dot(h3.reshape(K * MR, 128), w2,
                    preferred_element_type=jnp.float32)
        g3 = g.reshape(K, MR, 128)
        niota = lax.broadcasted_iota(jnp.int32, (K, MR, 128), 0)
        return jnp.max(jnp.where(niota < kpref, g3, -jnp.inf), axis=0)

    outs = (
        branch(w1_ref[...], c1_ref[...], b1_ref[...], w2_ref[...], 16),
        branch(wa_ref[...], wa_ref[...], ba_ref[...], wb_ref[...], 8),
        branch(wc_ref[...], wc_ref[...], bc_ref[...], wd_ref[...], 16),
        branch(we_ref[...], we_ref[...], be_ref[...], wf_ref[...], 24),
    )
    for i, o in enumerate(outs):
        out_ref[0, i * 128:(i + 1) * 128, :] = jnp.transpose(
            o + b2_ref[i:i + 1, :])


def _run_mlp(nb, ce, weights, b2all):
    grid = (ROWS // MR,)
    wspec = pl.BlockSpec((TW, 128), lambda t: (0, 0))
    w2spec = pl.BlockSpec((128, 128), lambda t: (0, 0))
    bspec = pl.BlockSpec((1, 128), lambda t: (0, 0))
    spb = S // MR
    return pl.pallas_call(
        _mlp_body,
        grid=grid,
        in_specs=[
            pl.BlockSpec((K, MR, TW), lambda t: (0, t, 0)),
            pl.BlockSpec((MR, TW), lambda t: (t, 0)),
            wspec, wspec, bspec, w2spec,
            wspec, bspec, w2spec,
            wspec, bspec, w2spec,
            wspec, bspec, w2spec,
            pl.BlockSpec((4, 128), lambda t: (0, 0)),
        ],
        out_specs=pl.BlockSpec((1, 512, MR),
                               lambda t: (t // spb, 0, t % spb)),
        out_shape=jax.ShapeDtypeStruct((B, 512, S), jnp.float32),
    )(nb, ce, *weights, b2all)


def kernel(l0_xyz, l0_points, sa_W1, sa_b1, sa_W2, sa_b2,
           ms_W1_0, ms_b1_0, ms_W2_0, ms_b2_0,
           ms_W1_1, ms_b1_1, ms_W2_1, ms_b2_1,
           ms_W1_2, ms_b1_2, ms_W2_2, ms_b2_2):
    x = l0_xyz[:, 0, :]
    y = l0_xyz[:, 1, :]
    z = l0_xyz[:, 2, :]

    cid, cx, cy, cz = _run_fps(x, y, z)

    def padT(a):
        return jnp.pad(a.T, ((0, 0), (0, 128 - B)))

    x3 = x.reshape(B, NLC, 128)
    y3 = y.reshape(B, NLC, 128)
    z3 = z.reshape(B, NLC, 128)
    idx24 = _run_topk(x3, y3, z3, padT(cx), padT(cy), padT(cz))

    pts = jnp.transpose(l0_points, (0, 2, 1)).reshape(B * N, 3)
    xyzt = jnp.transpose(l0_xyz, (0, 2, 1)).reshape(B * N, 3)
    table = jnp.concatenate(
        [xyzt, pts, jnp.zeros((B * N, TW - 6), jnp.float32)], axis=1)

    boff = (jnp.arange(B, dtype=jnp.int32) * N)[:, None, None]
    gnb = jnp.transpose(idx24 + boff, (2, 0, 1)).reshape(K * ROWS)
    gce = (cid + boff[:, :, 0]).reshape(ROWS)

    nb_flat, ce = _gather_rows(table, gnb, gce)
    nb = nb_flat.reshape(K, ROWS, TW)

    def padw1(w, r0):
        out = jnp.zeros((TW, 128), jnp.float32)
        return lax.dynamic_update_slice(out, w, (r0, 0))

    w1_base = padw1(sa_W1, 0)
    c1_base = padw1(sa_W1[:3], 0)
    w2_base = jnp.zeros((128, 128), jnp.float32).at[:64].set(sa_W2)
    b1_base = jnp.zeros((1, 128), jnp.float32).at[0, :64].set(sa_b1)

    def branch_w(W1, b1, W2):
        return (padw1(W1, 0),
                jnp.zeros((1, 128), jnp.float32).at[0, :64].set(b1),
                jnp.zeros((128, 128), jnp.float32).at[:64].set(W2))

    wa, ba, wb = branch_w(ms_W1_0, ms_b1_0, ms_W2_0)
    wc, bc, wd = branch_w(ms_W1_1, ms_b1_1, ms_W2_1)
    we, be, wf = branch_w(ms_W1_2, ms_b1_2, ms_W2_2)

    b2all = jnp.stack([sa_b2, ms_b2_0, ms_b2_1, ms_b2_2], axis=0)
    feats = _run_mlp(
        nb, ce,
        (w1_base, c1_base, b1_base, w2_base,
         wa, ba, wb, wc, bc, wd, we, be, wf),
        b2all)

    keypoints = jnp.stack([cx, cy, cz], axis=1)
    return (keypoints, feats)
